# Initial kernel scaffold; baseline (speedup 1.0000x reference)
import jax
import jax.numpy as jnp
from jax.experimental import pallas as pl


def kernel(x, edge_index, intra_enrichment_weights, W, att_src, att_dst, bias, enrichment_scale):
    raise NotImplementedError("write your pallas kernel here")


# speedup vs baseline: 62.1892x; 62.1892x over previous
import functools

import jax
import jax.numpy as jnp
from jax import lax
from jax.experimental import pallas as pl
from jax.experimental.pallas import tpu as pltpu
from jax.experimental.pallas import tpu_sc as plsc

N = 10000
E = 320000
D = 128
H = 2
C = 32
HC = H * C
NPAD = 10240
NC = 2
NS = 16
NW = NC * NS
B = 128
EPAD = 323584
EPW = EPAD // NW
NB = EPW // B
PW = 80
WCW = 16
RPW = NPAD // NS


def _lrelu(v):
    return jnp.where(v > 0, v, v * jnp.float32(0.2))


_DNUMS = lax.GatherDimensionNumbers(offset_dims=(), collapsed_slice_dims=(0,),
                                    start_index_map=(0,))


def _bcast(vec, lane):
    idx = jnp.full((16, 1), lane, jnp.int32)
    return lax.gather(vec, idx, _DNUMS, (1,),
                      mode=lax.GatherScatterMode.PROMISE_IN_BOUNDS)


def _dense_body(x_ref, w_ref, m_ref, xt_ref, a_ref):
    h = x_ref[...]
    h = jnp.where(h > 0, h, jnp.exp(h) - 1.0)
    xt = jnp.dot(h, w_ref[...], preferred_element_type=jnp.float32)
    xt_ref[...] = xt
    a_ref[...] = jnp.dot(xt, m_ref[...], preferred_element_type=jnp.float32)


def _dense(xpad, W, M):
    blk = 1024
    return pl.pallas_call(
        _dense_body,
        grid=(NPAD // blk,),
        in_specs=[
            pl.BlockSpec((blk, D), lambda i: (i, 0)),
            pl.BlockSpec((D, HC), lambda i: (0, 0)),
            pl.BlockSpec((HC, 4), lambda i: (0, 0)),
        ],
        out_specs=[
            pl.BlockSpec((blk, HC), lambda i: (i, 0)),
            pl.BlockSpec((blk, 4), lambda i: (i, 0)),
        ],
        out_shape=[
            jax.ShapeDtypeStruct((NPAD, HC), jnp.float32),
            jax.ShapeDtypeStruct((NPAD, 4), jnp.float32),
        ],
    )(xpad, W, M)


def _sc_body(src_hbm, dst_hbm, w_hbm, a_hbm, xt_hbm, z80, z16,
             acc_out, wc_out,
             a_t, rowsA, P, wc, srcv, dstv, wv, ex0b, ex1b,
             acc_sh, wcacc_sh, sem):
    cid = lax.axis_index("c")
    sid = lax.axis_index("s")
    wid = sid * NC + cid

    iota16 = lax.broadcasted_iota(jnp.int32, (16,), 0)
    zeros16 = jnp.zeros((16,), jnp.float32)
    onehot1 = (iota16 == 1).astype(jnp.float32)
    c64 = jnp.full((16,), 64, jnp.int32)
    c65 = jnp.full((16,), 65, jnp.int32)

    for r in range(B):
        P[r, pl.ds(64, 16)] = zeros16
        wc[r, pl.ds(0, 16)] = onehot1

    pltpu.sync_copy(z80, acc_sh.at[pl.ds(sid * RPW, RPW)])
    pltpu.sync_copy(z16, wcacc_sh.at[pl.ds(sid * RPW, RPW)])
    pltpu.sync_copy(a_hbm, a_t)
    plsc.subcore_barrier()

    def batch(g, carry):
        base = wid * EPW + g * B
        pltpu.sync_copy(src_hbm.at[pl.ds(base, B)], srcv)
        pltpu.sync_copy(dst_hbm.at[pl.ds(base, B)], dstv)
        pltpu.sync_copy(w_hbm.at[pl.ds(base, B)], wv)
        cp = pltpu.async_copy(xt_hbm.at[srcv], rowsA, sem)
        for i in range(B // 16):
            sl = pl.ds(i * 16, 16)
            s4 = srcv[sl] * 4
            d4 = dstv[sl] * 4
            e0 = jnp.exp(_lrelu(plsc.load_gather(a_t, [s4])
                                + plsc.load_gather(a_t, [d4 + 2])))
            e1 = jnp.exp(_lrelu(plsc.load_gather(a_t, [s4 + 1])
                                + plsc.load_gather(a_t, [d4 + 3])))
            ex0b[sl] = e0
            ex1b[sl] = e1
            rowid = iota16 + i * 16
            plsc.store_scatter(P, [rowid, c64], e0)
            plsc.store_scatter(P, [rowid, c65], e1)
            plsc.store_scatter(wc, [rowid, jnp.zeros((16,), jnp.int32)],
                               wv[sl])
        cp.wait()
        for i in range(B // 16):
            e0c = ex0b[pl.ds(i * 16, 16)]
            e1c = ex1b[pl.ds(i * 16, 16)]
            for l in range(16):
                e = i * 16 + l
                w0 = _bcast(e0c, l)
                w1 = _bcast(e1c, l)
                P[e, pl.ds(0, 16)] = rowsA[e, pl.ds(0, 16)] * w0
                P[e, pl.ds(16, 16)] = rowsA[e, pl.ds(16, 16)] * w0
                P[e, pl.ds(32, 16)] = rowsA[e, pl.ds(32, 16)] * w1
                P[e, pl.ds(48, 16)] = rowsA[e, pl.ds(48, 16)] * w1
        pltpu.sync_copy(P, acc_sh.at[dstv], add=True)
        pltpu.sync_copy(wc, wcacc_sh.at[srcv], add=True)
        return carry

    lax.fori_loop(0, NB, batch, 0)

    plsc.subcore_barrier()
    off = sid * RPW
    pltpu.sync_copy(acc_sh.at[pl.ds(off, RPW)],
                    acc_out.at[pl.ds(cid * NPAD + off, RPW)])
    pltpu.sync_copy(wcacc_sh.at[pl.ds(off, RPW)],
                    wc_out.at[pl.ds(cid * NPAD + off, RPW)])


def _sc_phase(src, dst, wpad, a_flat, xt, z80, z16):
    mesh = plsc.VectorSubcoreMesh(core_axis_name="c", subcore_axis_name="s",
                                  num_cores=NC, num_subcores=NS)
    f = pl.kernel(
        _sc_body, mesh=mesh,
        compiler_params=pltpu.CompilerParams(needs_layout_passes=False,
                                             use_tc_tiling_on_sc=False),
        out_type=[
            jax.ShapeDtypeStruct((NC * NPAD, PW), jnp.float32),
            jax.ShapeDtypeStruct((NC * NPAD, WCW), jnp.float32),
        ],
        scratch_types=[
            pltpu.VMEM((NPAD * 4,), jnp.float32),
            pltpu.VMEM((B, HC), jnp.float32),
            pltpu.VMEM((B, PW), jnp.float32),
            pltpu.VMEM((B, WCW), jnp.float32),
            pltpu.VMEM((B,), jnp.int32),
            pltpu.VMEM((B,), jnp.int32),
            pltpu.VMEM((B,), jnp.float32),
            pltpu.VMEM((B,), jnp.float32),
            pltpu.VMEM((B,), jnp.float32),
            pltpu.VMEM_SHARED((NPAD, PW), jnp.float32),
            pltpu.VMEM_SHARED((NPAD, WCW), jnp.float32),
            pltpu.SemaphoreType.DMA,
        ],
    )
    return f(src, dst, wpad, a_flat, xt, z80, z16)


def _final_body(acc_ref, wc_ref, xt_ref, a_ref, bias_ref, sf_ref, out_ref):
    a = a_ref[...]
    ex0 = jnp.exp(_lrelu(a[:, 0:1] + a[:, 2:3]))
    ex1 = jnp.exp(_lrelu(a[:, 1:2] + a[:, 3:4]))
    acc0 = acc_ref[0]
    acc1 = acc_ref[1]
    xtb = xt_ref[...]
    den0 = acc0[:, 64:65] + acc1[:, 64:65] + ex0 + jnp.float32(1e-16)
    den1 = acc0[:, 65:66] + acc1[:, 65:66] + ex1 + jnp.float32(1e-16)
    num = acc0[:, :HC] + acc1[:, :HC]
    num0 = num[:, :C] + ex0 * xtb[:, :C]
    num1 = num[:, C:] + ex1 * xtb[:, C:]
    intra = jnp.concatenate([num0 / den0, num1 / den1], axis=1) + bias_ref[...]
    wc = wc_ref[0] + wc_ref[1]
    nw = jnp.clip(wc[:, 0:1] / jnp.maximum(wc[:, 1:2], 1.0),
                  jnp.float32(0.3), jnp.float32(3.0))
    out_ref[...] = intra * (1.0 + sf_ref[0, 0] * (nw - 1.0))


def _final(acc, wcacc, xt, a, bias2d, sf):
    blk = 128
    return pl.pallas_call(
        _final_body,
        grid=(NPAD // blk,),
        in_specs=[
            pl.BlockSpec((NC, blk, PW), lambda i: (0, i, 0)),
            pl.BlockSpec((NC, blk, WCW), lambda i: (0, i, 0)),
            pl.BlockSpec((blk, HC), lambda i: (i, 0)),
            pl.BlockSpec((blk, 4), lambda i: (i, 0)),
            pl.BlockSpec((1, HC), lambda i: (0, 0)),
            pl.BlockSpec(memory_space=pltpu.SMEM),
        ],
        out_specs=pl.BlockSpec((blk, HC), lambda i: (i, 0)),
        out_shape=jax.ShapeDtypeStruct((NPAD, HC), jnp.float32),
    )(acc, wcacc, xt, a, bias2d, sf)


def kernel(x, edge_index, intra_enrichment_weights, W, att_src, att_dst,
           bias, enrichment_scale):
    f32 = jnp.float32
    M = jnp.zeros((HC, 4), f32)
    M = M.at[:C, 0].set(att_src[0, 0]).at[C:, 1].set(att_src[0, 1])
    M = M.at[:C, 2].set(att_dst[0, 0]).at[C:, 3].set(att_dst[0, 1])

    xpad = jnp.concatenate([x, jnp.zeros((NPAD - N, D), f32)], axis=0)
    xt, a = _dense(xpad, W, M)
    a_flat = a.reshape(NPAD * 4)

    pad_i = jnp.full((EPAD - E,), N, jnp.int32)
    src = jnp.concatenate([edge_index[0], pad_i])
    dst = jnp.concatenate([edge_index[1], pad_i])
    wpad = jnp.concatenate([intra_enrichment_weights,
                            jnp.zeros((EPAD - E,), f32)])
    z80 = jnp.zeros((RPW, PW), f32)
    z16 = jnp.zeros((RPW, WCW), f32)

    acc, wcacc = _sc_phase(src, dst, wpad, a_flat, xt, z80, z16)

    out = _final(acc.reshape(NC, NPAD, PW), wcacc.reshape(NC, NPAD, WCW),
                 xt, a, bias.reshape(1, HC),
                 (0.3 * jnp.tanh(enrichment_scale)).reshape(1, 1)
                 .astype(f32))
    return out[:N]
